# baseline (speedup 1.0000x reference)
import jax
import jax.numpy as jnp
from jax import lax
from jax.experimental import pallas as pl
from jax.experimental.pallas import tpu as pltpu
from jax.experimental.pallas import tpu_sc as plsc

N_SPARSE = 26
VOCAB = 100000
DIM = 32
BATCH = 4096
HIST_LEN = 50

NUM_CORES = 2
NUM_SUBCORES = 16
NUM_WORKERS = NUM_CORES * NUM_SUBCORES
ROWS_PER_WORKER = BATCH // NUM_WORKERS
SPARSE_COLS = N_SPARSE * DIM


def _row_mean_body(wt_ref, o_ref):
    o_ref[...] = jnp.sum(wt_ref[...], axis=0) * (1.0 / DIM)


def _row_mean(w_hist):
    return pl.pallas_call(
        _row_mean_body,
        out_shape=jax.ShapeDtypeStruct((VOCAB,), jnp.float32),
    )(w_hist.T)


def _sc_body(tbl_ref, sidx_ref, hist_ref, rm_ref, out_ref, hout_ref,
             idx_v, rows_v, hidx_v, hvals_v, sem):
    c = lax.axis_index("c")
    s = lax.axis_index("s")
    wid = s * NUM_CORES + c
    base = wid * ROWS_PER_WORKER

    for i in range(N_SPARSE):
        pltpu.sync_copy(sidx_ref.at[pl.ds(i * BATCH + base, ROWS_PER_WORKER)],
                        idx_v)
        pltpu.async_copy(tbl_ref.at[idx_v], rows_v, sem).wait()
        pltpu.sync_copy(
            rows_v,
            out_ref.at[pl.ds(base, ROWS_PER_WORKER), pl.ds(i * DIM, DIM)],
        )

    nh = ROWS_PER_WORKER * HIST_LEN
    pltpu.sync_copy(hist_ref.at[pl.ds(base * HIST_LEN, nh)], hidx_v)
    pltpu.async_copy(rm_ref.at[hidx_v], hvals_v, sem).wait()
    pltpu.sync_copy(hvals_v, hout_ref.at[pl.ds(base * HIST_LEN, nh)])


def kernel(sparse_0, sparse_1, sparse_2, sparse_3, sparse_4, sparse_5,
           sparse_6, sparse_7, sparse_8, sparse_9, sparse_10, sparse_11,
           sparse_12, sparse_13, sparse_14, sparse_15, sparse_16, sparse_17,
           sparse_18, sparse_19, sparse_20, sparse_21, sparse_22, sparse_23,
           sparse_24, sparse_25, hist,
           W_0, W_1, W_2, W_3, W_4, W_5, W_6, W_7, W_8, W_9, W_10, W_11,
           W_12, W_13, W_14, W_15, W_16, W_17, W_18, W_19, W_20, W_21,
           W_22, W_23, W_24, W_25, W_hist):
    sparse = [sparse_0, sparse_1, sparse_2, sparse_3, sparse_4, sparse_5,
              sparse_6, sparse_7, sparse_8, sparse_9, sparse_10, sparse_11,
              sparse_12, sparse_13, sparse_14, sparse_15, sparse_16,
              sparse_17, sparse_18, sparse_19, sparse_20, sparse_21,
              sparse_22, sparse_23, sparse_24, sparse_25]
    tables = [W_0, W_1, W_2, W_3, W_4, W_5, W_6, W_7, W_8, W_9, W_10, W_11,
              W_12, W_13, W_14, W_15, W_16, W_17, W_18, W_19, W_20, W_21,
              W_22, W_23, W_24, W_25]

    big_table = jnp.concatenate(tables, axis=0)
    sidx = jnp.concatenate(
        [x.reshape(BATCH) + (i * VOCAB) for i, x in enumerate(sparse)])
    rm = _row_mean(W_hist)

    mesh = plsc.VectorSubcoreMesh(core_axis_name="c", subcore_axis_name="s")
    sc = pl.kernel(
        _sc_body,
        out_type=(
            jax.ShapeDtypeStruct((BATCH, SPARSE_COLS), jnp.float32),
            jax.ShapeDtypeStruct((BATCH * HIST_LEN,), jnp.float32),
        ),
        mesh=mesh,
        compiler_params=pltpu.CompilerParams(use_tc_tiling_on_sc=False),
        scratch_types=[
            pltpu.VMEM((ROWS_PER_WORKER,), jnp.int32),
            pltpu.VMEM((ROWS_PER_WORKER, DIM), jnp.float32),
            pltpu.VMEM((ROWS_PER_WORKER * HIST_LEN,), jnp.int32),
            pltpu.VMEM((ROWS_PER_WORKER * HIST_LEN,), jnp.float32),
            pltpu.SemaphoreType.DMA,
        ],
    )
    out_sparse, out_hist = sc(big_table, sidx, hist.reshape(-1), rm)
    return jnp.concatenate(
        [out_sparse, out_hist.reshape(BATCH, HIST_LEN)], axis=-1)


# speedup vs baseline: 3.1042x; 3.1042x over previous
import jax
import jax.numpy as jnp
from jax import lax
from jax.experimental import pallas as pl
from jax.experimental.pallas import tpu as pltpu
from jax.experimental.pallas import tpu_sc as plsc

N_SPARSE = 26
VOCAB = 100000
DIM = 32
BATCH = 4096
HIST_LEN = 50

NUM_CORES = 2
NUM_SUBCORES = 16
NUM_WORKERS = NUM_CORES * NUM_SUBCORES
ROWS_PER_WORKER = BATCH // NUM_WORKERS
OUT_COLS = N_SPARSE * DIM + HIST_LEN


def _row_mean_body(w_ref, o_ref):
    o_ref[...] = jnp.sum(w_ref[...], axis=1, keepdims=True) * (1.0 / DIM)


def _row_mean(w_hist):
    blk = 10000
    out = pl.pallas_call(
        _row_mean_body,
        grid=(VOCAB // blk,),
        in_specs=[pl.BlockSpec((blk, DIM), lambda i: (i, 0))],
        out_specs=pl.BlockSpec((blk, 1), lambda i: (i, 0)),
        out_shape=jax.ShapeDtypeStruct((VOCAB, 1), jnp.float32),
    )(w_hist)
    return out.reshape(VOCAB)


def _sc_body(*refs):
    tables = refs[:N_SPARSE]
    sidx_ref = refs[N_SPARSE]
    hist_ref = refs[N_SPARSE + 1]
    rm_ref = refs[N_SPARSE + 2]
    out_ref = refs[N_SPARSE + 3]
    idx_v, dvals_v, hidx_v, hvals_v, sem = refs[N_SPARSE + 4:]

    c = lax.axis_index("c")
    s = lax.axis_index("s")
    wid = s * NUM_CORES + c
    base = wid * ROWS_PER_WORKER

    for i in range(N_SPARSE):
        tbl = tables[i]
        pltpu.sync_copy(sidx_ref.at[pl.ds(i * BATCH + base, ROWS_PER_WORKER)],
                        idx_v)

        def _fire(d, _, tbl=tbl):
            pltpu.async_copy(tbl.at[d].at[idx_v], dvals_v.at[d], sem)
            return 0

        lax.fori_loop(0, DIM, _fire, 0)
        pltpu.make_async_copy(tbl.at[:, pl.ds(0, ROWS_PER_WORKER)],
                              dvals_v, sem).wait()
        pltpu.sync_copy(
            dvals_v,
            out_ref.at[pl.ds(i * DIM, DIM), pl.ds(base, ROWS_PER_WORKER)],
        )

    pltpu.sync_copy(hist_ref.at[:, pl.ds(base, ROWS_PER_WORKER)], hidx_v)

    def _fire_h(l, _):
        pltpu.async_copy(rm_ref.at[hidx_v.at[l]], hvals_v.at[l], sem)
        return 0

    lax.fori_loop(0, HIST_LEN, _fire_h, 0)
    pltpu.make_async_copy(hist_ref.at[:, pl.ds(0, ROWS_PER_WORKER)],
                          hvals_v, sem).wait()
    pltpu.sync_copy(
        hvals_v,
        out_ref.at[pl.ds(N_SPARSE * DIM, HIST_LEN),
                   pl.ds(base, ROWS_PER_WORKER)],
    )


def kernel(sparse_0, sparse_1, sparse_2, sparse_3, sparse_4, sparse_5,
           sparse_6, sparse_7, sparse_8, sparse_9, sparse_10, sparse_11,
           sparse_12, sparse_13, sparse_14, sparse_15, sparse_16, sparse_17,
           sparse_18, sparse_19, sparse_20, sparse_21, sparse_22, sparse_23,
           sparse_24, sparse_25, hist,
           W_0, W_1, W_2, W_3, W_4, W_5, W_6, W_7, W_8, W_9, W_10, W_11,
           W_12, W_13, W_14, W_15, W_16, W_17, W_18, W_19, W_20, W_21,
           W_22, W_23, W_24, W_25, W_hist):
    sparse = [sparse_0, sparse_1, sparse_2, sparse_3, sparse_4, sparse_5,
              sparse_6, sparse_7, sparse_8, sparse_9, sparse_10, sparse_11,
              sparse_12, sparse_13, sparse_14, sparse_15, sparse_16,
              sparse_17, sparse_18, sparse_19, sparse_20, sparse_21,
              sparse_22, sparse_23, sparse_24, sparse_25]
    tables = [W_0, W_1, W_2, W_3, W_4, W_5, W_6, W_7, W_8, W_9, W_10, W_11,
              W_12, W_13, W_14, W_15, W_16, W_17, W_18, W_19, W_20, W_21,
              W_22, W_23, W_24, W_25]

    sidx = jnp.concatenate([x.reshape(BATCH) for x in sparse])
    rm = _row_mean(W_hist)

    mesh = plsc.VectorSubcoreMesh(core_axis_name="c", subcore_axis_name="s")
    sc = pl.kernel(
        _sc_body,
        out_type=jax.ShapeDtypeStruct((OUT_COLS, BATCH), jnp.float32),
        mesh=mesh,
        compiler_params=pltpu.CompilerParams(use_tc_tiling_on_sc=False),
        scratch_types=[
            pltpu.VMEM((ROWS_PER_WORKER,), jnp.int32),
            pltpu.VMEM((DIM, ROWS_PER_WORKER), jnp.float32),
            pltpu.VMEM((HIST_LEN, ROWS_PER_WORKER), jnp.int32),
            pltpu.VMEM((HIST_LEN, ROWS_PER_WORKER), jnp.float32),
            pltpu.SemaphoreType.DMA,
        ],
    )
    out_t = sc(*[w.T for w in tables], sidx, hist.T, rm)
    return out_t.T


# speedup vs baseline: 3.6862x; 1.1875x over previous
import jax
import jax.numpy as jnp
from jax import lax
from jax.experimental import pallas as pl
from jax.experimental.pallas import tpu as pltpu
from jax.experimental.pallas import tpu_sc as plsc

N_SPARSE = 26
VOCAB = 100000
DIM = 32
BATCH = 4096
HIST_LEN = 50

NUM_CORES = 2
NUM_SUBCORES = 16
NUM_WORKERS = NUM_CORES * NUM_SUBCORES
ROWS_PER_WORKER = BATCH // NUM_WORKERS
OUT_COLS = N_SPARSE * DIM + HIST_LEN


def _row_mean_body(w_ref, o_ref):
    o_ref[...] = jnp.sum(w_ref[...], axis=1, keepdims=True) * (1.0 / DIM)


def _row_mean(w_hist):
    blk = 10000
    out = pl.pallas_call(
        _row_mean_body,
        grid=(VOCAB // blk,),
        in_specs=[pl.BlockSpec((blk, DIM), lambda i: (i, 0))],
        out_specs=pl.BlockSpec((blk, 1), lambda i: (i, 0)),
        out_shape=jax.ShapeDtypeStruct((VOCAB, 1), jnp.float32),
    )(w_hist)
    return out.reshape(VOCAB)


def _make_sparse_body(n_tables):
    def _body(*refs):
        tables = refs[:n_tables]
        sidx_ref = refs[n_tables]
        out_ref = refs[n_tables + 1]
        idx_v, dvals_v, sem = refs[n_tables + 2:]

        c = lax.axis_index("c")
        s = lax.axis_index("s")
        wid = s * NUM_CORES + c
        base = wid * ROWS_PER_WORKER

        for i in range(n_tables):
            tbl = tables[i]
            pltpu.sync_copy(
                sidx_ref.at[pl.ds(i * BATCH + base, ROWS_PER_WORKER)], idx_v)

            def _fire(d, _, tbl=tbl):
                pltpu.async_copy(tbl.at[d].at[idx_v], dvals_v.at[d], sem)
                return 0

            lax.fori_loop(0, DIM, _fire, 0)
            pltpu.make_async_copy(tbl.at[:, pl.ds(0, ROWS_PER_WORKER)],
                                  dvals_v, sem).wait()
            pltpu.sync_copy(
                dvals_v,
                out_ref.at[pl.ds(i * DIM, DIM), pl.ds(base, ROWS_PER_WORKER)],
            )

    return _body


def _hist_body(hist_ref, rm_ref, out_ref, hidx_v, hvals_v, sem):
    c = lax.axis_index("c")
    s = lax.axis_index("s")
    wid = s * NUM_CORES + c
    base = wid * ROWS_PER_WORKER

    pltpu.sync_copy(hist_ref.at[:, pl.ds(base, ROWS_PER_WORKER)], hidx_v)

    def _fire_h(l, _):
        pltpu.async_copy(rm_ref.at[hidx_v.at[l]], hvals_v.at[l], sem)
        return 0

    lax.fori_loop(0, HIST_LEN, _fire_h, 0)
    pltpu.make_async_copy(hist_ref.at[:, pl.ds(0, ROWS_PER_WORKER)],
                          hvals_v, sem).wait()
    pltpu.sync_copy(hvals_v, out_ref.at[:, pl.ds(base, ROWS_PER_WORKER)])


def kernel(sparse_0, sparse_1, sparse_2, sparse_3, sparse_4, sparse_5,
           sparse_6, sparse_7, sparse_8, sparse_9, sparse_10, sparse_11,
           sparse_12, sparse_13, sparse_14, sparse_15, sparse_16, sparse_17,
           sparse_18, sparse_19, sparse_20, sparse_21, sparse_22, sparse_23,
           sparse_24, sparse_25, hist,
           W_0, W_1, W_2, W_3, W_4, W_5, W_6, W_7, W_8, W_9, W_10, W_11,
           W_12, W_13, W_14, W_15, W_16, W_17, W_18, W_19, W_20, W_21,
           W_22, W_23, W_24, W_25, W_hist):
    sparse = [sparse_0, sparse_1, sparse_2, sparse_3, sparse_4, sparse_5,
              sparse_6, sparse_7, sparse_8, sparse_9, sparse_10, sparse_11,
              sparse_12, sparse_13, sparse_14, sparse_15, sparse_16,
              sparse_17, sparse_18, sparse_19, sparse_20, sparse_21,
              sparse_22, sparse_23, sparse_24, sparse_25]
    tables = [W_0, W_1, W_2, W_3, W_4, W_5, W_6, W_7, W_8, W_9, W_10, W_11,
              W_12, W_13, W_14, W_15, W_16, W_17, W_18, W_19, W_20, W_21,
              W_22, W_23, W_24, W_25]

    rm = _row_mean(W_hist)

    mesh = plsc.VectorSubcoreMesh(core_axis_name="c", subcore_axis_name="s")

    groups = [(0, 7), (7, 14), (14, 20), (20, 26)]
    pieces = []
    for lo, hi in groups:
        n = hi - lo
        sidx_g = jnp.concatenate(
            [sparse[i].reshape(BATCH) for i in range(lo, hi)])
        sc = pl.kernel(
            _make_sparse_body(n),
            out_type=jax.ShapeDtypeStruct((n * DIM, BATCH), jnp.float32),
            mesh=mesh,
            compiler_params=pltpu.CompilerParams(use_tc_tiling_on_sc=False),
            scratch_types=[
                pltpu.VMEM((ROWS_PER_WORKER,), jnp.int32),
                pltpu.VMEM((DIM, ROWS_PER_WORKER), jnp.float32),
                pltpu.SemaphoreType.DMA,
            ],
        )
        pieces.append(sc(*[tables[i].T for i in range(lo, hi)], sidx_g))

    sc_h = pl.kernel(
        _hist_body,
        out_type=jax.ShapeDtypeStruct((HIST_LEN, BATCH), jnp.float32),
        mesh=mesh,
        compiler_params=pltpu.CompilerParams(use_tc_tiling_on_sc=False),
        scratch_types=[
            pltpu.VMEM((HIST_LEN, ROWS_PER_WORKER), jnp.int32),
            pltpu.VMEM((HIST_LEN, ROWS_PER_WORKER), jnp.float32),
            pltpu.SemaphoreType.DMA,
        ],
    )
    pieces.append(sc_h(hist.T, rm))

    out_t = jnp.concatenate(pieces, axis=0)
    return out_t.T
